# baseline (speedup 1.0000x reference)
import functools

import jax
import jax.numpy as jnp
from jax import lax
from jax.experimental import pallas as pl
from jax.experimental.pallas import tpu as pltpu
from jax.experimental.pallas import tpu_sc as plsc

C = 128
DEG = 32
NW = 32
TILE_NODES = 320
SUB = 128
NSUB = 3
TILE_EXT = SUB * NSUB
N_PAD = NW * TILE_NODES


def _matmul_kernel(x_ref, w_ref, o_ref):
  o_ref[...] = jnp.dot(x_ref[...], w_ref[...], preferred_element_type=jnp.float32)


def _gru_kernel(s_ref, x_ref, wih_ref, whh_ref, bih_ref, bhh_ref, o_ref):
  s = s_ref[...]
  h = x_ref[...]
  dn = (((1,), (1,)), ((), ()))
  gi = lax.dot_general(s, wih_ref[...], dn, preferred_element_type=jnp.float32)
  gi = gi + bih_ref[...]
  gh = lax.dot_general(h, whh_ref[...], dn, preferred_element_type=jnp.float32)
  gh = gh + bhh_ref[...]
  r = jax.nn.sigmoid(gi[:, :C] + gh[:, :C])
  z = jax.nn.sigmoid(gi[:, C:2 * C] + gh[:, C:2 * C])
  n = jnp.tanh(gi[:, 2 * C:] + r * gh[:, 2 * C:])
  o_ref[...] = (1.0 - z) * n + z * h


def _gather_sum_body(table_hbm, sidx_hbm, out_hbm, idx_v, acc_v, sem):
  c = lax.axis_index("c")
  s = lax.axis_index("s")
  wid = s * 2 + c
  pltpu.sync_copy(sidx_hbm.at[wid], idx_v)
  for sub in range(NSUB):
    pltpu.async_copy(table_hbm.at[idx_v.at[sub * DEG]],
                     acc_v.at[pl.ds(sub * SUB, SUB)], sem)
  for sub in range(NSUB):
    pltpu.make_async_copy(table_hbm.at[idx_v.at[sub * DEG]],
                          acc_v.at[pl.ds(sub * SUB, SUB)], sem).wait()

  def fire(d, carry):
    for sub in range(NSUB):
      pltpu.async_copy(table_hbm.at[idx_v.at[sub * DEG + d]],
                       acc_v.at[pl.ds(sub * SUB, SUB)], sem, add=True)
    return carry

  lax.fori_loop(1, DEG, fire, 0)

  def drain(d, carry):
    for sub in range(NSUB):
      pltpu.make_async_copy(table_hbm.at[idx_v.at[sub * DEG + d]],
                            acc_v.at[pl.ds(sub * SUB, SUB)], sem).wait()
    return carry

  lax.fori_loop(1, DEG, drain, 0)
  pltpu.sync_copy(acc_v.at[pl.ds(0, TILE_NODES)],
                  out_hbm.at[pl.ds(wid * TILE_NODES, TILE_NODES)])


def _make_gather_sum():
  mesh = plsc.VectorSubcoreMesh(core_axis_name="c", subcore_axis_name="s")
  return pl.kernel(
      _gather_sum_body,
      out_type=jax.ShapeDtypeStruct((N_PAD, C), jnp.float32),
      mesh=mesh,
      scratch_types=[
          pltpu.VMEM((NSUB * DEG, SUB), jnp.int32),
          pltpu.VMEM((TILE_EXT, C), jnp.float32),
          pltpu.SemaphoreType.DMA,
      ],
  )


@jax.jit
def kernel(x, edge_index, weight, W_ih, W_hh, b_ih, b_hh):
  n = x.shape[0]
  x_pad = jnp.zeros((N_PAD, C), jnp.float32).at[:n].set(x)
  e = edge_index.astype(jnp.int32)
  e_pad = jnp.full((N_PAD, DEG), n, jnp.int32).at[:n].set(e)
  e_t = e_pad.reshape(NW, TILE_NODES, DEG)
  e_t = jnp.concatenate(
      [e_t, jnp.full((NW, TILE_EXT - TILE_NODES, DEG), n, jnp.int32)], axis=1)
  sidx = e_t.reshape(NW, NSUB, SUB, DEG).transpose(0, 1, 3, 2).reshape(
      NW, NSUB * DEG, SUB)

  bm = 512
  m_pad = pl.pallas_call(
      _matmul_kernel,
      grid=(N_PAD // bm,),
      in_specs=[
          pl.BlockSpec((bm, C), lambda i: (i, 0)),
          pl.BlockSpec((C, C), lambda i: (0, 0)),
      ],
      out_specs=pl.BlockSpec((bm, C), lambda i: (i, 0)),
      out_shape=jax.ShapeDtypeStruct((N_PAD, C), jnp.float32),
  )(x_pad, weight[0])

  s_pad = _make_gather_sum()(m_pad, sidx)

  out = pl.pallas_call(
      _gru_kernel,
      grid=(N_PAD // bm,),
      in_specs=[
          pl.BlockSpec((bm, C), lambda i: (i, 0)),
          pl.BlockSpec((bm, C), lambda i: (i, 0)),
          pl.BlockSpec((3 * C, C), lambda i: (0, 0)),
          pl.BlockSpec((3 * C, C), lambda i: (0, 0)),
          pl.BlockSpec((1, 3 * C), lambda i: (0, 0)),
          pl.BlockSpec((1, 3 * C), lambda i: (0, 0)),
      ],
      out_specs=pl.BlockSpec((bm, C), lambda i: (i, 0)),
      out_shape=jax.ShapeDtypeStruct((N_PAD, C), jnp.float32),
  )(s_pad, x_pad, W_ih, W_hh, b_ih.reshape(1, 3 * C), b_hh.reshape(1, 3 * C))

  return out[:n]


# speedup vs baseline: 14.5390x; 14.5390x over previous
import jax
import jax.numpy as jnp
from jax import lax
from jax.experimental import pallas as pl
from jax.experimental.pallas import tpu as pltpu
from jax.experimental.pallas import tpu_sc as plsc

C = 128
DEG = 32
NW = 32
CPT = C // NW
G = 512
NB = 20
N_PAD = NB * G
L = 16


def _matmul_t_kernel(w_ref, x_ref, o_ref):
  o_ref[...] = lax.dot_general(w_ref[...], x_ref[...],
                               (((0,), (0,)), ((), ())),
                               preferred_element_type=jnp.float32)


def _gru_t_kernel(s_ref, x_ref, wih_ref, whh_ref, bih_ref, bhh_ref, o_ref):
  s = s_ref[...]
  h = x_ref[...]
  dn = (((1,), (0,)), ((), ()))
  gi = lax.dot_general(wih_ref[...], s, dn,
                       preferred_element_type=jnp.float32) + bih_ref[...]
  gh = lax.dot_general(whh_ref[...], h, dn,
                       preferred_element_type=jnp.float32) + bhh_ref[...]
  r = jax.nn.sigmoid(gi[:C] + gh[:C])
  z = jax.nn.sigmoid(gi[C:2 * C] + gh[C:2 * C])
  n = jnp.tanh(gi[2 * C:] + r * gh[2 * C:])
  o_ref[...] = (1.0 - z) * n + z * h


EBLK = G * DEG


def _gather_sum_body(mt_hbm, e_hbm, out_hbm, tab_v, ebuf_v, obuf_v,
                     sem_t, sem_e, sem_o):
  cix = lax.axis_index("c")
  six = lax.axis_index("s")
  wid = six * 2 + cix
  c0 = wid * CPT

  for cc in range(CPT):
    pltpu.async_copy(mt_hbm.at[c0 + cc], tab_v.at[pl.ds(cc * N_PAD, N_PAD)],
                     sem_t)
  pltpu.async_copy(e_hbm.at[0], ebuf_v.at[pl.ds(0, EBLK)], sem_e)
  for cc in range(CPT):
    pltpu.make_async_copy(mt_hbm.at[c0 + cc],
                          tab_v.at[pl.ds(cc * N_PAD, N_PAD)], sem_t).wait()

  offc = [jnp.full((L,), cc * N_PAD, jnp.int32) for cc in range(CPT)]

  def do_block(b, k):
    pltpu.make_async_copy(e_hbm.at[b], ebuf_v.at[pl.ds(k * EBLK, EBLK)],
                          sem_e).wait()

    @pl.when(b + 1 < NB)
    def _():
      pltpu.async_copy(e_hbm.at[b + 1],
                       ebuf_v.at[pl.ds((1 - k) * EBLK, EBLK)], sem_e)

    def g_body(g, carry):
      ebase = k * EBLK + g * (DEG * L)
      obase = k * (CPT * G) + g * L
      acc = [jnp.zeros((L,), jnp.float32) for _ in range(CPT)]
      for d in range(DEG):
        idx = ebuf_v[pl.ds(ebase + d * L, L)]
        for cc in range(CPT):
          acc[cc] = acc[cc] + plsc.load_gather(tab_v, [idx + offc[cc]])
      for cc in range(CPT):
        obuf_v[pl.ds(obase + cc * G, L)] = acc[cc]
      return carry

    lax.fori_loop(0, G // L, g_body, 0)
    for cc in range(CPT):
      pltpu.async_copy(obuf_v.at[pl.ds(k * (CPT * G) + cc * G, G)],
                       out_hbm.at[c0 + cc, pl.ds(b * G, G)], sem_o)

  def pair(bb, carry):
    for k in range(2):
      b = bb * 2 + k

      @pl.when(bb > 0)
      def _():
        for cc in range(CPT):
          pltpu.make_async_copy(
              obuf_v.at[pl.ds(k * (CPT * G) + cc * G, G)],
              out_hbm.at[c0 + cc, pl.ds(b * G, G)], sem_o).wait()

      do_block(b, k)
    return carry

  lax.fori_loop(0, NB // 2, pair, 0)
  for k in range(2):
    b = NB - 2 + k
    for cc in range(CPT):
      pltpu.make_async_copy(
          obuf_v.at[pl.ds(k * (CPT * G) + cc * G, G)],
          out_hbm.at[c0 + cc, pl.ds(b * G, G)], sem_o).wait()


def _make_gather_sum():
  mesh = plsc.VectorSubcoreMesh(core_axis_name="c", subcore_axis_name="s")
  return pl.kernel(
      _gather_sum_body,
      out_type=jax.ShapeDtypeStruct((C, N_PAD), jnp.float32),
      mesh=mesh,
      scratch_types=[
          pltpu.VMEM((CPT * N_PAD,), jnp.float32),
          pltpu.VMEM((2 * EBLK,), jnp.int32),
          pltpu.VMEM((2 * CPT * G,), jnp.float32),
          pltpu.SemaphoreType.DMA,
          pltpu.SemaphoreType.DMA,
          pltpu.SemaphoreType.DMA,
      ],
      compiler_params=pltpu.CompilerParams(needs_layout_passes=False),
  )


@jax.jit
def kernel(x, edge_index, weight, W_ih, W_hh, b_ih, b_hh):
  n = x.shape[0]
  x_t = jnp.zeros((C, N_PAD), jnp.float32).at[:, :n].set(x.T)
  e = edge_index.astype(jnp.int32)
  e_pad = jnp.full((N_PAD, DEG), n, jnp.int32).at[:n].set(e)
  e_blk = e_pad.reshape(NB, G // L, L, DEG).transpose(0, 1, 3, 2).reshape(
      NB, EBLK)
  bih_bc = jnp.broadcast_to(b_ih[:, None], (3 * C, G))
  bhh_bc = jnp.broadcast_to(b_hh[:, None], (3 * C, G))

  m_t = pl.pallas_call(
      _matmul_t_kernel,
      grid=(NB,),
      in_specs=[
          pl.BlockSpec((C, C), lambda i: (0, 0)),
          pl.BlockSpec((C, G), lambda i: (0, i)),
      ],
      out_specs=pl.BlockSpec((C, G), lambda i: (0, i)),
      out_shape=jax.ShapeDtypeStruct((C, N_PAD), jnp.float32),
  )(weight[0], x_t)

  s_t = _make_gather_sum()(m_t, e_blk)

  out_t = pl.pallas_call(
      _gru_t_kernel,
      grid=(NB,),
      in_specs=[
          pl.BlockSpec((C, G), lambda i: (0, i)),
          pl.BlockSpec((C, G), lambda i: (0, i)),
          pl.BlockSpec((3 * C, C), lambda i: (0, 0)),
          pl.BlockSpec((3 * C, C), lambda i: (0, 0)),
          pl.BlockSpec((3 * C, G), lambda i: (0, 0)),
          pl.BlockSpec((3 * C, G), lambda i: (0, 0)),
      ],
      out_specs=pl.BlockSpec((C, G), lambda i: (0, i)),
      out_shape=jax.ShapeDtypeStruct((C, N_PAD), jnp.float32),
  )(s_t, x_t, W_ih, W_hh, bih_bc, bhh_bc)

  return out_t[:, :n].T


# speedup vs baseline: 15.4535x; 1.0629x over previous
import jax
import jax.numpy as jnp
from jax import lax
from jax.experimental import pallas as pl
from jax.experimental.pallas import tpu as pltpu
from jax.experimental.pallas import tpu_sc as plsc

C = 128
DEG = 32
NW = 32
CPT = C // NW
G = 512
NB = 20
N_PAD = NB * G
L = 16


def _matmul_t_kernel(w_ref, x_ref, o_ref):
  o_ref[...] = lax.dot_general(w_ref[...], x_ref[...],
                               (((0,), (1,)), ((), ())),
                               preferred_element_type=jnp.float32)


def _gru_kernel(st_ref, x_ref, wih_ref, whh_ref, bih_ref, bhh_ref, o_ref):
  h = x_ref[...]
  gi = lax.dot_general(st_ref[...], wih_ref[...], (((0,), (1,)), ((), ())),
                       preferred_element_type=jnp.float32) + bih_ref[...]
  gh = lax.dot_general(h, whh_ref[...], (((1,), (1,)), ((), ())),
                       preferred_element_type=jnp.float32) + bhh_ref[...]
  r = jax.nn.sigmoid(gi[:, :C] + gh[:, :C])
  z = jax.nn.sigmoid(gi[:, C:2 * C] + gh[:, C:2 * C])
  n = jnp.tanh(gi[:, 2 * C:] + r * gh[:, 2 * C:])
  o_ref[...] = (1.0 - z) * n + z * h


EBLK = G * DEG


def _gather_sum_body(mt_hbm, e_hbm, out_hbm, tab_v, ebuf_v, obuf_v,
                     sem_t, sem_e, sem_o):
  cix = lax.axis_index("c")
  six = lax.axis_index("s")
  wid = six * 2 + cix
  c0 = wid * CPT

  for cc in range(CPT):
    pltpu.async_copy(mt_hbm.at[c0 + cc], tab_v.at[pl.ds(cc * N_PAD, N_PAD)],
                     sem_t)
  pltpu.async_copy(e_hbm.at[0], ebuf_v.at[pl.ds(0, EBLK)], sem_e)
  for cc in range(CPT):
    pltpu.make_async_copy(mt_hbm.at[c0 + cc],
                          tab_v.at[pl.ds(cc * N_PAD, N_PAD)], sem_t).wait()

  offc = [jnp.full((L,), cc * N_PAD, jnp.int32) for cc in range(CPT)]

  def do_block(b, k):
    pltpu.make_async_copy(e_hbm.at[b], ebuf_v.at[pl.ds(k * EBLK, EBLK)],
                          sem_e).wait()

    @pl.when(b + 1 < NB)
    def _():
      pltpu.async_copy(e_hbm.at[b + 1],
                       ebuf_v.at[pl.ds((1 - k) * EBLK, EBLK)], sem_e)

    def g_body(g, carry):
      ebase = k * EBLK + g * (DEG * L)
      obase = k * (CPT * G) + g * L
      acc = [jnp.zeros((L,), jnp.float32) for _ in range(CPT)]
      for d in range(DEG):
        idx = ebuf_v[pl.ds(ebase + d * L, L)]
        for cc in range(CPT):
          acc[cc] = acc[cc] + plsc.load_gather(tab_v, [idx + offc[cc]])
      for cc in range(CPT):
        obuf_v[pl.ds(obase + cc * G, L)] = acc[cc]
      return carry

    lax.fori_loop(0, G // L, g_body, 0)
    for cc in range(CPT):
      pltpu.async_copy(obuf_v.at[pl.ds(k * (CPT * G) + cc * G, G)],
                       out_hbm.at[c0 + cc, pl.ds(b * G, G)], sem_o)

  def pair(bb, carry):
    for k in range(2):
      b = bb * 2 + k

      @pl.when(bb > 0)
      def _():
        for cc in range(CPT):
          pltpu.make_async_copy(
              obuf_v.at[pl.ds(k * (CPT * G) + cc * G, G)],
              out_hbm.at[c0 + cc, pl.ds(b * G, G)], sem_o).wait()

      do_block(b, k)
    return carry

  lax.fori_loop(0, NB // 2, pair, 0)
  for k in range(2):
    b = NB - 2 + k
    for cc in range(CPT):
      pltpu.make_async_copy(
          obuf_v.at[pl.ds(k * (CPT * G) + cc * G, G)],
          out_hbm.at[c0 + cc, pl.ds(b * G, G)], sem_o).wait()


def _make_gather_sum():
  mesh = plsc.VectorSubcoreMesh(core_axis_name="c", subcore_axis_name="s")
  return pl.kernel(
      _gather_sum_body,
      out_type=jax.ShapeDtypeStruct((C, N_PAD), jnp.float32),
      mesh=mesh,
      scratch_types=[
          pltpu.VMEM((CPT * N_PAD,), jnp.float32),
          pltpu.VMEM((2 * EBLK,), jnp.int32),
          pltpu.VMEM((2 * CPT * G,), jnp.float32),
          pltpu.SemaphoreType.DMA,
          pltpu.SemaphoreType.DMA,
          pltpu.SemaphoreType.DMA,
      ],
      compiler_params=pltpu.CompilerParams(needs_layout_passes=False),
  )


@jax.jit
def kernel(x, edge_index, weight, W_ih, W_hh, b_ih, b_hh):
  n = x.shape[0]
  x_pad = jnp.zeros((N_PAD, C), jnp.float32).at[:n].set(x)
  e = edge_index.astype(jnp.int32)
  e_pad = jnp.full((N_PAD, DEG), n, jnp.int32).at[:n].set(e)
  e_blk = e_pad.reshape(NB, G // L, L, DEG).transpose(0, 1, 3, 2).reshape(
      NB, EBLK)

  m_t = pl.pallas_call(
      _matmul_t_kernel,
      grid=(NB,),
      in_specs=[
          pl.BlockSpec((C, C), lambda i: (0, 0)),
          pl.BlockSpec((G, C), lambda i: (i, 0)),
      ],
      out_specs=pl.BlockSpec((C, G), lambda i: (0, i)),
      out_shape=jax.ShapeDtypeStruct((C, N_PAD), jnp.float32),
  )(weight[0], x_pad)

  s_t = _make_gather_sum()(m_t, e_blk)

  out = pl.pallas_call(
      _gru_kernel,
      grid=(NB,),
      in_specs=[
          pl.BlockSpec((C, G), lambda i: (0, i)),
          pl.BlockSpec((G, C), lambda i: (i, 0)),
          pl.BlockSpec((3 * C, C), lambda i: (0, 0)),
          pl.BlockSpec((3 * C, C), lambda i: (0, 0)),
          pl.BlockSpec((1, 3 * C), lambda i: (0, 0)),
          pl.BlockSpec((1, 3 * C), lambda i: (0, 0)),
      ],
      out_specs=pl.BlockSpec((G, C), lambda i: (i, 0)),
      out_shape=jax.ShapeDtypeStruct((N_PAD, C), jnp.float32),
  )(s_t, x_pad, W_ih, W_hh, b_ih.reshape(1, 3 * C), b_hh.reshape(1, 3 * C))

  return out[:n]


# speedup vs baseline: 16.4536x; 1.0647x over previous
import jax
import jax.numpy as jnp
from jax import lax
from jax.experimental import pallas as pl
from jax.experimental.pallas import tpu as pltpu
from jax.experimental.pallas import tpu_sc as plsc

C = 128
DEG = 32
NW = 32
CPT = C // NW
G = 1024
NB = 10
N_PAD = NB * G
L = 16


def _matmul_t_kernel(w_ref, x_ref, o_ref):
  o_ref[...] = lax.dot_general(w_ref[...], x_ref[...],
                               (((0,), (1,)), ((), ())),
                               preferred_element_type=jnp.float32)


def _gru_kernel(st_ref, x_ref, wih_ref, whh_ref, bih_ref, bhh_ref, o_ref):
  h = x_ref[...]
  gi = lax.dot_general(st_ref[...], wih_ref[...], (((0,), (1,)), ((), ())),
                       preferred_element_type=jnp.float32) + bih_ref[...]
  gh = lax.dot_general(h, whh_ref[...], (((1,), (1,)), ((), ())),
                       preferred_element_type=jnp.float32) + bhh_ref[...]
  r = jax.nn.sigmoid(gi[:, :C] + gh[:, :C])
  z = jax.nn.sigmoid(gi[:, C:2 * C] + gh[:, C:2 * C])
  n = jnp.tanh(gi[:, 2 * C:] + r * gh[:, 2 * C:])
  o_ref[...] = (1.0 - z) * n + z * h


EBLK = G * DEG


def _gather_sum_body(mt_hbm, e_hbm, out_hbm, tab_v, ebuf_v, obuf_v,
                     sem_t, sem_e, sem_o):
  cix = lax.axis_index("c")
  six = lax.axis_index("s")
  wid = six * 2 + cix
  c0 = wid * CPT

  for cc in range(CPT):
    pltpu.async_copy(mt_hbm.at[c0 + cc], tab_v.at[pl.ds(cc * N_PAD, N_PAD)],
                     sem_t)
  pltpu.async_copy(e_hbm.at[0], ebuf_v.at[pl.ds(0, EBLK)], sem_e)
  for cc in range(CPT):
    pltpu.make_async_copy(mt_hbm.at[c0 + cc],
                          tab_v.at[pl.ds(cc * N_PAD, N_PAD)], sem_t).wait()

  offc = [jnp.full((L,), cc * N_PAD, jnp.int32) for cc in range(CPT)]

  def do_block(b, k):
    pltpu.make_async_copy(e_hbm.at[b], ebuf_v.at[pl.ds(k * EBLK, EBLK)],
                          sem_e).wait()

    @pl.when(b + 1 < NB)
    def _():
      pltpu.async_copy(e_hbm.at[b + 1],
                       ebuf_v.at[pl.ds((1 - k) * EBLK, EBLK)], sem_e)

    def g_body(g, carry):
      ebase = k * EBLK + g * (DEG * L)
      obase = k * (CPT * G) + g * L
      idx0 = ebuf_v[pl.ds(ebase, L)]
      acc = [plsc.load_gather(tab_v, [idx0 + offc[cc]]) for cc in range(CPT)]
      for d in range(1, DEG):
        idx = ebuf_v[pl.ds(ebase + d * L, L)]
        for cc in range(CPT):
          acc[cc] = acc[cc] + plsc.load_gather(tab_v, [idx + offc[cc]])
      for cc in range(CPT):
        obuf_v[pl.ds(obase + cc * G, L)] = acc[cc]
      return carry

    lax.fori_loop(0, G // L, g_body, 0)
    for cc in range(CPT):
      pltpu.async_copy(obuf_v.at[pl.ds(k * (CPT * G) + cc * G, G)],
                       out_hbm.at[c0 + cc, pl.ds(b * G, G)], sem_o)

  def pair(bb, carry):
    for k in range(2):
      b = bb * 2 + k

      @pl.when(bb > 0)
      def _():
        for cc in range(CPT):
          pltpu.make_async_copy(
              obuf_v.at[pl.ds(k * (CPT * G) + cc * G, G)],
              out_hbm.at[c0 + cc, pl.ds(b * G, G)], sem_o).wait()

      do_block(b, k)
    return carry

  lax.fori_loop(0, NB // 2, pair, 0)
  for k in range(2):
    b = NB - 2 + k
    for cc in range(CPT):
      pltpu.make_async_copy(
          obuf_v.at[pl.ds(k * (CPT * G) + cc * G, G)],
          out_hbm.at[c0 + cc, pl.ds(b * G, G)], sem_o).wait()


def _make_gather_sum():
  mesh = plsc.VectorSubcoreMesh(core_axis_name="c", subcore_axis_name="s")
  return pl.kernel(
      _gather_sum_body,
      out_type=jax.ShapeDtypeStruct((C, N_PAD), jnp.float32),
      mesh=mesh,
      scratch_types=[
          pltpu.VMEM((CPT * N_PAD,), jnp.float32),
          pltpu.VMEM((2 * EBLK,), jnp.int32),
          pltpu.VMEM((2 * CPT * G,), jnp.float32),
          pltpu.SemaphoreType.DMA,
          pltpu.SemaphoreType.DMA,
          pltpu.SemaphoreType.DMA,
      ],
      compiler_params=pltpu.CompilerParams(needs_layout_passes=False),
  )


@jax.jit
def kernel(x, edge_index, weight, W_ih, W_hh, b_ih, b_hh):
  n = x.shape[0]
  x_pad = jnp.zeros((N_PAD, C), jnp.float32).at[:n].set(x)
  e = edge_index.astype(jnp.int32)
  e_pad = jnp.full((N_PAD, DEG), n, jnp.int32).at[:n].set(e)
  e_blk = e_pad.reshape(NB, G // L, L, DEG).transpose(0, 1, 3, 2).reshape(
      NB, EBLK)

  m_t = pl.pallas_call(
      _matmul_t_kernel,
      grid=(NB,),
      in_specs=[
          pl.BlockSpec((C, C), lambda i: (0, 0)),
          pl.BlockSpec((G, C), lambda i: (i, 0)),
      ],
      out_specs=pl.BlockSpec((C, G), lambda i: (0, i)),
      out_shape=jax.ShapeDtypeStruct((C, N_PAD), jnp.float32),
  )(weight[0], x_pad)

  s_t = _make_gather_sum()(m_t, e_blk)

  out = pl.pallas_call(
      _gru_kernel,
      grid=(NB,),
      in_specs=[
          pl.BlockSpec((C, G), lambda i: (0, i)),
          pl.BlockSpec((G, C), lambda i: (i, 0)),
          pl.BlockSpec((3 * C, C), lambda i: (0, 0)),
          pl.BlockSpec((3 * C, C), lambda i: (0, 0)),
          pl.BlockSpec((1, 3 * C), lambda i: (0, 0)),
          pl.BlockSpec((1, 3 * C), lambda i: (0, 0)),
      ],
      out_specs=pl.BlockSpec((G, C), lambda i: (i, 0)),
      out_shape=jax.ShapeDtypeStruct((N_PAD, C), jnp.float32),
  )(s_t, x_pad, W_ih, W_hh, b_ih.reshape(1, 3 * C), b_hh.reshape(1, 3 * C))

  return out[:n]


# speedup vs baseline: 20.7557x; 1.2615x over previous
import jax
import jax.numpy as jnp
from jax import lax
from jax.experimental import pallas as pl
from jax.experimental.pallas import tpu as pltpu
from jax.experimental.pallas import tpu_sc as plsc

C = 128
DEG = 32
NW = 32
CPT = C // NW
G = 1024
NB = 10
N_PAD = NB * G
L = 16


def _matmul_t_kernel(w_ref, x_ref, o_ref):
  m = lax.dot_general(w_ref[...], x_ref[...], (((0,), (1,)), ((), ())),
                      preferred_element_type=jnp.float32)
  hi = lax.convert_element_type(
      lax.convert_element_type(m[:C // 2], jnp.bfloat16), jnp.float32)
  lo = lax.convert_element_type(
      lax.convert_element_type(m[C // 2:], jnp.bfloat16), jnp.float32)
  u_hi = lax.bitcast_convert_type(hi, jnp.uint32)
  u_lo = lax.bitcast_convert_type(lo, jnp.uint32) >> 16
  o_ref[...] = lax.bitcast_convert_type(u_hi | u_lo, jnp.int32)


def _gru_kernel(st_ref, x_ref, wih_ref, whh_ref, bih_ref, bhh_ref, o_ref):
  h = x_ref[...]
  gi = lax.dot_general(st_ref[...], wih_ref[...], (((0,), (1,)), ((), ())),
                       preferred_element_type=jnp.float32) + bih_ref[...]
  gh = lax.dot_general(h, whh_ref[...], (((1,), (1,)), ((), ())),
                       preferred_element_type=jnp.float32) + bhh_ref[...]
  r = jax.nn.sigmoid(gi[:, :C] + gh[:, :C])
  z = jax.nn.sigmoid(gi[:, C:2 * C] + gh[:, C:2 * C])
  n = jnp.tanh(gi[:, 2 * C:] + r * gh[:, 2 * C:])
  o_ref[...] = (1.0 - z) * n + z * h


EBLK = G * DEG


PPT = 2


def _gather_sum_body(mt_hbm, e_hbm, out_hbm, tab_v, ebuf_v, obuf_v,
                     sem_t, sem_e, sem_o):
  cix = lax.axis_index("c")
  six = lax.axis_index("s")
  wid = six * 2 + cix
  p0 = wid * PPT
  orow = [p0, p0 + C // 2, p0 + 1, p0 + 1 + C // 2]

  for r in range(PPT):
    pltpu.async_copy(mt_hbm.at[p0 + r], tab_v.at[pl.ds(r * N_PAD, N_PAD)],
                     sem_t)
  pltpu.async_copy(e_hbm.at[0], ebuf_v.at[pl.ds(0, EBLK)], sem_e)
  for r in range(PPT):
    pltpu.make_async_copy(mt_hbm.at[p0 + r],
                          tab_v.at[pl.ds(r * N_PAD, N_PAD)], sem_t).wait()

  offr = [jnp.full((L,), r * N_PAD, jnp.int32) for r in range(PPT)]
  mhi = jnp.full((L,), -65536, jnp.int32)
  sh16 = jnp.full((L,), 16, jnp.int32)

  def do_block(b, k):
    pltpu.make_async_copy(e_hbm.at[b], ebuf_v.at[pl.ds(k * EBLK, EBLK)],
                          sem_e).wait()

    @pl.when(b + 1 < NB)
    def _():
      pltpu.async_copy(e_hbm.at[b + 1],
                       ebuf_v.at[pl.ds((1 - k) * EBLK, EBLK)], sem_e)

    def g_body(g, carry):
      ebase = k * EBLK + g * (DEG * L)
      obase = k * (CPT * G) + g * L
      acc = [jnp.zeros((L,), jnp.float32) for _ in range(2 * PPT)]
      for d in range(DEG):
        idx = ebuf_v[pl.ds(ebase + d * L, L)]
        for r in range(PPT):
          gv = plsc.load_gather(tab_v, [idx + offr[r]])
          acc[2 * r] = acc[2 * r] + plsc.bitcast(gv & mhi, jnp.float32)
          acc[2 * r + 1] = acc[2 * r + 1] + plsc.bitcast(
              lax.shift_left(gv, sh16), jnp.float32)
      for j in range(2 * PPT):
        obuf_v[pl.ds(obase + j * G, L)] = acc[j]
      return carry

    lax.fori_loop(0, G // L, g_body, 0)
    for j in range(2 * PPT):
      pltpu.async_copy(obuf_v.at[pl.ds(k * (CPT * G) + j * G, G)],
                       out_hbm.at[orow[j], pl.ds(b * G, G)], sem_o)

  def pair(bb, carry):
    for k in range(2):
      b = bb * 2 + k

      @pl.when(bb > 0)
      def _():
        for j in range(2 * PPT):
          pltpu.make_async_copy(
              obuf_v.at[pl.ds(k * (CPT * G) + j * G, G)],
              out_hbm.at[orow[j], pl.ds(b * G, G)], sem_o).wait()

      do_block(b, k)
    return carry

  lax.fori_loop(0, NB // 2, pair, 0)
  for k in range(2):
    b = NB - 2 + k
    for j in range(2 * PPT):
      pltpu.make_async_copy(
          obuf_v.at[pl.ds(k * (CPT * G) + j * G, G)],
          out_hbm.at[orow[j], pl.ds(b * G, G)], sem_o).wait()


def _make_gather_sum():
  mesh = plsc.VectorSubcoreMesh(core_axis_name="c", subcore_axis_name="s")
  return pl.kernel(
      _gather_sum_body,
      out_type=jax.ShapeDtypeStruct((C, N_PAD), jnp.float32),
      mesh=mesh,
      scratch_types=[
          pltpu.VMEM((PPT * N_PAD,), jnp.int32),
          pltpu.VMEM((2 * EBLK,), jnp.int32),
          pltpu.VMEM((2 * CPT * G,), jnp.float32),
          pltpu.SemaphoreType.DMA,
          pltpu.SemaphoreType.DMA,
          pltpu.SemaphoreType.DMA,
      ],
      compiler_params=pltpu.CompilerParams(needs_layout_passes=False),
  )


@jax.jit
def kernel(x, edge_index, weight, W_ih, W_hh, b_ih, b_hh):
  n = x.shape[0]
  x_pad = jnp.zeros((N_PAD, C), jnp.float32).at[:n].set(x)
  e = edge_index.astype(jnp.int32)
  e_pad = jnp.full((N_PAD, DEG), n, jnp.int32).at[:n].set(e)
  e_blk = e_pad.reshape(NB, G // L, L, DEG).transpose(0, 1, 3, 2).reshape(
      NB, EBLK)

  m_t = pl.pallas_call(
      _matmul_t_kernel,
      grid=(NB,),
      in_specs=[
          pl.BlockSpec((C, C), lambda i: (0, 0)),
          pl.BlockSpec((G, C), lambda i: (i, 0)),
      ],
      out_specs=pl.BlockSpec((C // 2, G), lambda i: (0, i)),
      out_shape=jax.ShapeDtypeStruct((C // 2, N_PAD), jnp.int32),
  )(weight[0], x_pad)

  s_t = _make_gather_sum()(m_t, e_blk)

  out = pl.pallas_call(
      _gru_kernel,
      grid=(NB,),
      in_specs=[
          pl.BlockSpec((C, G), lambda i: (0, i)),
          pl.BlockSpec((G, C), lambda i: (i, 0)),
          pl.BlockSpec((3 * C, C), lambda i: (0, 0)),
          pl.BlockSpec((3 * C, C), lambda i: (0, 0)),
          pl.BlockSpec((1, 3 * C), lambda i: (0, 0)),
          pl.BlockSpec((1, 3 * C), lambda i: (0, 0)),
      ],
      out_specs=pl.BlockSpec((G, C), lambda i: (i, 0)),
      out_shape=jax.ShapeDtypeStruct((N_PAD, C), jnp.float32),
  )(s_t, x_pad, W_ih, W_hh, b_ih.reshape(1, 3 * C), b_hh.reshape(1, 3 * C))

  return out[:n]
